# Initial kernel scaffold; baseline (speedup 1.0000x reference)
import jax
import jax.numpy as jnp
from jax.experimental import pallas as pl


def kernel(points, img, kernel):
    raise NotImplementedError("write your pallas kernel here")


# speedup vs baseline: 1.3120x; 1.3120x over previous
import functools

import jax
import jax.numpy as jnp
from jax import lax
from jax.experimental import pallas as pl
from jax.experimental.pallas import tpu as pltpu, tpu_sc as plsc

H = 384
W = 384
N = 64
KSZ = 11
RAD = 5
NC = 2
NS = 16
NW = NC * NS
SPW = N // NW
SAMPLE = H * W
ZCHUNK = 24576
NZ = SAMPLE // ZCHUNK
STRIP = KSZ * W
NVREG = 8
_INT_MIN = -(2**31)


def _body(pts_hbm, kern_hbm, out_hbm, pts_v, kern_v, zbuf, psem, ksem, zsem):
    c = lax.axis_index("c")
    s = lax.axis_index("s")
    w = s * NC + c

    cp_p = pltpu.async_copy(pts_hbm.at[w], pts_v, psem)
    cp_k = pltpu.async_copy(kern_hbm, kern_v, ksem)

    zero16 = jnp.zeros((16,), jnp.float32)

    def _zero(i, carry):
        zbuf[pl.ds(i * 16, 16)] = zero16
        return carry

    lax.fori_loop(0, ZCHUNK // 16, _zero, 0, unroll=8)

    cp_p.wait()
    cp_k.wait()

    lane = lax.broadcasted_iota(jnp.int32, (16,), 0)

    pv = pts_v[...]

    def _scalar_at(i):
        return jnp.clip(pv[i].astype(jnp.int32), 0, W - 1)

    kys, kxs, kms, kvs = [], [], [], []
    for j in range(NVREG):
        t = lane + j * 16
        ky = lax.div(t, KSZ)
        kys.append(ky)
        kxs.append(t - ky * KSZ)
        kms.append(t < KSZ * KSZ)
        kvs.append(kern_v[pl.ds(j * 16, 16)])

    for si in range(SPW):
        n = w * SPW + si
        cx = _scalar_at(2 * si)
        cy = _scalar_at(2 * si + 1)
        yw = jnp.clip(cy - RAD, 0, H - KSZ)
        base = n * SAMPLE

        zcopies = [
            pltpu.async_copy(
                zbuf, out_hbm.at[pl.ds(base + i * ZCHUNK, ZCHUNK)], zsem
            )
            for i in range(NZ)
        ]
        for cp in zcopies:
            cp.wait()

        idxs, masks, widxs, wmasks = [], [], [], []
        for j in range(NVREG):
            yy = kys[j] + (cy - RAD)
            xx = kxs[j] + (cx - RAD)
            xxw = jnp.where(xx < 0, xx + W, xx)
            m = kms[j] & (yy >= 0) & (yy < H) & (xx < W)
            idx = jnp.where(m, (yy - yw) * W + xxw, 0)
            idxs.append(idx)
            masks.append(m)
            m2 = kms[j] & (yy < 0) & (xx < W)
            idx2 = jnp.where(m2, (yy + KSZ) * W + xxw, 0)
            widxs.append(idx2)
            wmasks.append(m2)
            plsc.store_scatter(zbuf, [idx], kvs[j], mask=m)

        pltpu.sync_copy(
            zbuf.at[pl.ds(0, STRIP)],
            out_hbm.at[pl.ds(base + yw * W, STRIP)],
        )

        for j in range(NVREG):
            plsc.store_scatter(zbuf, [idxs[j]], zero16, mask=masks[j])

        @pl.when(cy < RAD)
        def _wrap_rows():
            for j in range(NVREG):
                plsc.store_scatter(zbuf, [widxs[j]], kvs[j], mask=wmasks[j])
            pltpu.sync_copy(
                zbuf.at[pl.ds(0, STRIP)],
                out_hbm.at[pl.ds(base + (H - KSZ) * W, STRIP)],
            )
            for j in range(NVREG):
                plsc.store_scatter(zbuf, [widxs[j]], zero16, mask=wmasks[j])


@jax.jit
def _heatmap_sc(pts32, kern128):
    mesh = plsc.VectorSubcoreMesh(
        core_axis_name="c", subcore_axis_name="s", num_cores=NC, num_subcores=NS
    )
    run = pl.kernel(
        _body,
        out_type=jax.ShapeDtypeStruct((N * H * W,), jnp.float32),
        mesh=mesh,
        scratch_types=[
            pltpu.VMEM((16,), jnp.float32),
            pltpu.VMEM((NVREG * 16,), jnp.float32),
            pltpu.VMEM((ZCHUNK,), jnp.float32),
            pltpu.SemaphoreType.DMA,
            pltpu.SemaphoreType.DMA,
            pltpu.SemaphoreType.DMA,
        ],
        compiler_params=pltpu.CompilerParams(needs_layout_passes=False),
    )
    return run(pts32, kern128)


def kernel(points, img, kernel):
    n, _, h, w = img.shape
    pts32 = jnp.pad(points.reshape(NW, 2 * SPW), ((0, 0), (0, 16 - 2 * SPW)))
    kern128 = jnp.pad(kernel.reshape(-1), (0, NVREG * 16 - KSZ * KSZ)).astype(
        jnp.float32
    )
    return _heatmap_sc(pts32, kern128).reshape(n, h, w)


# speedup vs baseline: 2.9034x; 2.2130x over previous
import jax
import jax.numpy as jnp
from jax import lax
from jax.experimental import pallas as pl
from jax.experimental.pallas import tpu as pltpu, tpu_sc as plsc

H = 384
W = 384
N = 64
KSZ = 11
RAD = 5
NC = 2
NS = 16
NW = NC * NS
SPW = N // NW
ZROWS = 64
NZ = H // ZROWS
SROWS = 24
NVREG = 8


def _body(pts_hbm, kern_hbm, out_hbm, pts_v, kern_v, zbuf, psem, ksem, zsem):
    c = lax.axis_index("c")
    s = lax.axis_index("s")
    w = s * NC + c

    cp_p = pltpu.async_copy(pts_hbm.at[w], pts_v, psem)
    cp_k = pltpu.async_copy(kern_hbm, kern_v, ksem)

    zero16 = jnp.zeros((16,), jnp.float32)

    def _zero_flat(i, carry):
        r = lax.div(i, W // 16)
        col = (i - r * (W // 16)) * 16
        zbuf[r, pl.ds(col, 16)] = zero16
        return carry

    lax.fori_loop(0, ZROWS * (W // 16), _zero_flat, 0, unroll=8)

    cp_p.wait()
    cp_k.wait()

    lane = lax.broadcasted_iota(jnp.int32, (16,), 0)
    pv = pts_v[...]

    def _scalar_at(i):
        return jnp.clip(pv[i].astype(jnp.int32), 0, W - 1)

    kys, kxs, kms, kvs = [], [], [], []
    for j in range(NVREG):
        t = lane + j * 16
        ky = lax.div(t, KSZ)
        kys.append(ky)
        kxs.append(t - ky * KSZ)
        kms.append(t < KSZ * KSZ)
        kvs.append(kern_v[pl.ds(j * 16, 16)])

    for si in range(SPW):
        n = w * SPW + si
        cx = _scalar_at(2 * si)
        cy = _scalar_at(2 * si + 1)
        yw = pl.multiple_of(jnp.clip(lax.div(cy - RAD, 8) * 8, 0, H - SROWS), 8)

        zcopies = [
            pltpu.async_copy(
                zbuf, out_hbm.at[n, pl.ds(i * ZROWS, ZROWS)], zsem
            )
            for i in range(NZ)
        ]
        for cp in zcopies:
            cp.wait()

        rows, cols, masks, wrows, wmasks = [], [], [], [], []
        for j in range(NVREG):
            yy = kys[j] + (cy - RAD)
            xx = kxs[j] + (cx - RAD)
            xxw = jnp.where(xx < 0, xx + W, xx)
            m = kms[j] & (yy >= 0) & (yy < H) & (xx < W)
            rows.append(jnp.where(m, yy - yw, 0))
            cols.append(jnp.where(m, xxw, 0))
            masks.append(m)
            m2 = kms[j] & (yy < 0) & (xx < W)
            wrows.append(jnp.where(m2, yy + SROWS, 0))
            wmasks.append(m2)
            plsc.store_scatter(zbuf, [rows[j], cols[j]], kvs[j], mask=m)

        pltpu.sync_copy(
            zbuf.at[pl.ds(0, SROWS)],
            out_hbm.at[n, pl.ds(yw, SROWS)],
        )

        for j in range(NVREG):
            plsc.store_scatter(zbuf, [rows[j], cols[j]], zero16, mask=masks[j])

        @pl.when(cy < RAD)
        def _wrap_rows():
            for j in range(NVREG):
                plsc.store_scatter(
                    zbuf, [wrows[j], cols[j]], kvs[j], mask=wmasks[j]
                )
            pltpu.sync_copy(
                zbuf.at[pl.ds(0, SROWS)],
                out_hbm.at[n, pl.ds(H - SROWS, SROWS)],
            )
            for j in range(NVREG):
                plsc.store_scatter(
                    zbuf, [wrows[j], cols[j]], zero16, mask=wmasks[j]
                )


@jax.jit
def _heatmap_sc(pts32, kern128):
    mesh = plsc.VectorSubcoreMesh(
        core_axis_name="c", subcore_axis_name="s", num_cores=NC, num_subcores=NS
    )
    run = pl.kernel(
        _body,
        out_type=jax.ShapeDtypeStruct((N, H, W), jnp.float32),
        mesh=mesh,
        scratch_types=[
            pltpu.VMEM((16,), jnp.float32),
            pltpu.VMEM((NVREG * 16,), jnp.float32),
            pltpu.VMEM((ZROWS, W), jnp.float32),
            pltpu.SemaphoreType.DMA,
            pltpu.SemaphoreType.DMA,
            pltpu.SemaphoreType.DMA,
        ],
        compiler_params=pltpu.CompilerParams(needs_layout_passes=False),
    )
    return run(pts32, kern128)


def kernel(points, img, kernel):
    pts32 = jnp.pad(points.reshape(NW, 2 * SPW), ((0, 0), (0, 16 - 2 * SPW)))
    kern128 = jnp.pad(kernel.reshape(-1), (0, NVREG * 16 - KSZ * KSZ)).astype(
        jnp.float32
    )
    return _heatmap_sc(pts32, kern128)
